# baseline (speedup 1.0000x reference)
import functools

import jax
import jax.numpy as jnp
from jax import lax
from jax.experimental import pallas as pl
from jax.experimental.pallas import tpu as pltpu
from jax.experimental.pallas import tpu_sc as plsc

_MERGED = 100
_TL = 128
_W = 512
_NB = 2


def _tc_body(m_ref, e_ref, lt_ref, x_ref, xt_ref, tok_ref):
    for i in range(_NB):
        _one_batch(i, m_ref, e_ref, lt_ref, x_ref, xt_ref, tok_ref)


def _one_batch(i, m_ref, e_ref, lt_ref, x_ref, xt_ref, tok_ref):
    x = x_ref[i]
    C, HW = x.shape
    FW = 32
    G = HW // 4
    L = HW - 3 * _MERGED
    LP = 768

    f32 = jnp.float32
    t_row = lax.broadcasted_iota(jnp.int32, (1, HW), 1)
    col = t_row % FW
    row = t_row // FW
    col_even = (col & 1) == 0
    row_even = (row & 1) == 0

    x1 = jnp.roll(x, 1, axis=1)
    a = jnp.where(col_even, x, x1)
    refv = jnp.where(row_even, a, jnp.roll(a, FW, axis=1))
    d = jnp.abs(x - refv)
    tsum = jnp.sum(d, axis=0, keepdims=True)

    u = tsum + jnp.roll(tsum, -1, axis=1)
    v = u + jnp.roll(u, -FW, axis=1)
    bf16 = jnp.bfloat16
    v_hi = v.astype(bf16)
    vr = v - v_hi.astype(f32)
    v_mid = vr.astype(bf16)
    v_lo = (vr - v_mid.astype(f32)).astype(bf16)
    E = e_ref[...]
    sdims = (((1,), (0,)), ((), ()))
    s = (lax.dot_general(v_hi, E, sdims, preferred_element_type=f32)
         + lax.dot_general(v_mid, E, sdims, preferred_element_type=f32)
         + lax.dot_general(v_lo, E, sdims, preferred_element_type=f32))

    S_g = jnp.broadcast_to(s, (G, G))
    S_j = jnp.transpose(S_g)
    j_i = lax.broadcasted_iota(jnp.int32, (G, G), 0)
    g_i = lax.broadcasted_iota(jnp.int32, (G, G), 1)
    cmp = (S_j < S_g) | ((S_j == S_g) & (j_i < g_i))
    rank = jnp.sum(cmp.astype(jnp.int32), axis=0, keepdims=True)
    keep_group = (rank >= _MERGED).astype(bf16)

    kgt = lax.dot_general(keep_group, m_ref[...], (((1,), (1,)), ((), ())),
                          preferred_element_type=f32)
    keep = (row_even & col_even) | (kgt > 0.5)
    keep_f = keep.astype(f32)

    pos = lax.dot_general(keep_f.astype(bf16), lt_ref[...],
                          (((1,), (0,)), ((), ())),
                          preferred_element_type=f32)

    ta = (t_row // 256).astype(bf16)
    tb = (t_row % 256).astype(bf16)
    dims = (((1,), (1,)), ((), ()))
    for k in range(LP // _TL):
        l0 = k * _TL
        t0 = min(l0, HW - _W)
        sl = slice(t0, t0 + _W)
        l_i = (l0 + lax.broadcasted_iota(jnp.int32, (_TL, _W), 0)).astype(f32)
        Pb = ((jnp.broadcast_to(pos[:, sl], (_TL, _W)) == l_i)
              & jnp.broadcast_to(keep[:, sl], (_TL, _W))).astype(bf16)
        tok_k = (256.0 * lax.dot_general(ta[:, sl], Pb, dims, preferred_element_type=f32)
                 + lax.dot_general(tb[:, sl], Pb, dims, preferred_element_type=f32))
        n = min(_TL, L - l0)
        tok_ref[i, :, l0:l0 + n] = tok_k[:, :n].astype(jnp.int32)

    xt_ref[i] = jnp.transpose(x)


def _tc_phase(x):
    B, C, FH, FW = x.shape
    HW = FH * FW
    G = HW // 4
    L = HW - 3 * _MERGED
    xf = x.reshape(B, C, HW)

    t = jnp.arange(HW, dtype=jnp.int32)
    g_of_t = (t // FW // 2) * (FW // 2) + (t % FW) // 2
    gi = jnp.arange(G, dtype=jnp.int32)[None, :]
    m_const = (g_of_t[:, None] == gi).astype(jnp.bfloat16)
    topleft_t = (t // FW // 2) * 2 * FW + ((t % FW) // 2) * 2
    e_const = ((t[:, None] == topleft_t[:, None])
               & (g_of_t[:, None] == gi)).astype(jnp.bfloat16)
    lt_const = (t[:, None] < t[None, :]).astype(jnp.bfloat16)

    xt, tok = pl.pallas_call(
        _tc_body,
        grid=(B // _NB,),
        in_specs=[
            pl.BlockSpec((HW, G), lambda b: (0, 0)),
            pl.BlockSpec((HW, G), lambda b: (0, 0)),
            pl.BlockSpec((HW, HW), lambda b: (0, 0)),
            pl.BlockSpec((_NB, C, HW), lambda b: (b, 0, 0)),
        ],
        out_specs=[
            pl.BlockSpec((_NB, HW, C), lambda b: (b, 0, 0)),
            pl.BlockSpec((_NB, 1, L), lambda b: (b, 0, 0)),
        ],
        out_shape=[
            jax.ShapeDtypeStruct((B, HW, C), jnp.float32),
            jax.ShapeDtypeStruct((B, 1, L), jnp.int32),
        ],
    )(m_const, e_const, lt_const, xf)
    return xt, tok.reshape(B, L)


def _sc_gather(table, gidx, n_rows, d):
    info = plsc.get_sparse_core_info()
    nc, ns = info.num_cores, info.num_subcores
    nw = nc * ns
    per_w = n_rows // nw
    chunk = 64
    nfull = per_w // chunk
    tail = per_w - nfull * chunk
    sizes = [chunk] * nfull + ([tail] if tail else [])
    offs = [j * chunk for j in range(len(sizes))]
    mesh = plsc.VectorSubcoreMesh(core_axis_name="c", subcore_axis_name="s")

    @functools.partial(
        pl.kernel, mesh=mesh,
        out_type=jax.ShapeDtypeStruct((n_rows, d), jnp.float32),
        scratch_types=[
            pltpu.VMEM((per_w,), jnp.int32),
            pltpu.VMEM((chunk, d), jnp.float32),
            pltpu.VMEM((chunk, d), jnp.float32),
            pltpu.SemaphoreType.DMA,
            pltpu.SemaphoreType.DMA,
        ],
    )
    def k(table_hbm, idx_hbm, out_hbm, idx_v, buf0, buf1, sem0, sem1):
        wid = lax.axis_index("s") * nc + lax.axis_index("c")
        base = wid * per_w
        pltpu.sync_copy(idx_hbm.at[pl.ds(base, per_w)], idx_v)
        bufs = (buf0, buf1)
        sems = (sem0, sem1)
        cops = [None] * len(sizes)
        cops[0] = pltpu.async_copy(
            table_hbm.at[idx_v.at[pl.ds(0, sizes[0])]],
            buf0.at[pl.ds(0, sizes[0])], sem0)
        for j in range(len(sizes)):
            if j + 1 < len(sizes):
                cops[j + 1] = pltpu.async_copy(
                    table_hbm.at[idx_v.at[pl.ds(offs[j + 1], sizes[j + 1])]],
                    bufs[(j + 1) % 2].at[pl.ds(0, sizes[j + 1])],
                    sems[(j + 1) % 2])
            cops[j].wait()
            pltpu.sync_copy(bufs[j % 2].at[pl.ds(0, sizes[j])],
                            out_hbm.at[pl.ds(base + offs[j], sizes[j])])

    return k(table, gidx)


def kernel(x):
    B, C, FH, FW = x.shape
    HW = FH * FW
    L = HW - 3 * _MERGED
    xt, tok = _tc_phase(x)
    gidx = (tok + (jnp.arange(B, dtype=jnp.int32) * HW)[:, None]).reshape(B * L)
    out = _sc_gather(xt.reshape(B * HW, C), gidx, B * L, C)
    return (out.reshape(B, L, C), tok)
